# Initial kernel scaffold; baseline (speedup 1.0000x reference)
import jax
import jax.numpy as jnp
from jax.experimental import pallas as pl


def kernel(user_emb, item_emb, edge_index, edge_weight):
    raise NotImplementedError("write your pallas kernel here")


# speedup vs baseline: 8.0681x; 8.0681x over previous
import functools

import jax
import jax.numpy as jnp
from jax import lax
from jax.experimental import pallas as pl
from jax.experimental.pallas import tpu as pltpu
from jax.experimental.pallas import tpu_sc as plsc

N_U = 25000
N_I = 25000
N = N_U + N_I
D = 64
E = 3200000 // 2
CHUNK = 128
G = E // CHUNK
GC = G // 2
NS = 16
ROWCH = 8
NROWCH = N_U // ROWCH
WOUT_IT = (NROWCH + NS - 1) // NS

_MESH = plsc.VectorSubcoreMesh(core_axis_name="c", subcore_axis_name="s")
_CPARAMS = pltpu.CompilerParams(needs_layout_passes=False,
                                use_tc_tiling_on_sc=False)


def _tile_range(c, s, per_core, n_tiles):
    base = per_core // n_tiles
    rem = per_core % n_tiles
    start = c * per_core + s * base + jnp.minimum(s, rem)
    cnt = base + (s < rem).astype(jnp.int32)
    return start, cnt


def _zero_zbuf(zbuf):
    z = jnp.zeros((16,), jnp.float32)
    for j in range(ROWCH):
        for d in range(D // 16):
            zbuf[j, pl.ds(d * 16, 16)] = z


DEGW = 16
DEGCH = 125


def _deg_body(rowloc2d, w2d, deg_out, idxbuf, wbuf, ubuf, tbuf, deg_sp):
    c = lax.axis_index("c")
    s = lax.axis_index("s")

    z = jnp.zeros((16,), jnp.float32)
    for j in range(DEGCH):
        tbuf[j, pl.ds(0, 16)] = z
    for j in range(13):
        cid = j * NS + s
        @pl.when(cid < 200)
        def _():
            pltpu.sync_copy(tbuf, deg_sp.at[pl.ds(cid * DEGCH, DEGCH)])
    for j in range(CHUNK):
        ubuf[j, pl.ds(0, 16)] = z
    plsc.subcore_barrier()

    start, cnt = _tile_range(c, s, GC, NS)
    iota = lax.iota(jnp.int32, 16)
    czero = jnp.zeros((16,), jnp.int32)

    def body(g, carry):
        pltpu.sync_copy(rowloc2d.at[g], idxbuf)
        pltpu.sync_copy(w2d.at[g], wbuf)
        for j in range(CHUNK // 16):
            plsc.store_scatter(ubuf, [iota + j * 16, czero], wbuf[pl.ds(j * 16, 16)])
        pltpu.sync_copy(ubuf, deg_sp.at[idxbuf], add=True)
        return carry

    lax.fori_loop(start, start + cnt, body, 0)
    plsc.subcore_barrier()

    for j in range(13):
        cid = j * NS + s
        @pl.when(cid < 200)
        def _():
            pltpu.sync_copy(deg_sp.at[pl.ds(cid * DEGCH, DEGCH)], tbuf)
            pltpu.sync_copy(tbuf, deg_out.at[pl.ds(c * N_U + cid * DEGCH, DEGCH)])


_deg_kernel = functools.partial(
    pl.kernel,
    out_type=jax.ShapeDtypeStruct((N, DEGW), jnp.float32),
    mesh=_MESH,
    compiler_params=_CPARAMS,
    scratch_types=[
        pltpu.VMEM((CHUNK,), jnp.int32),
        pltpu.VMEM((CHUNK,), jnp.float32),
        pltpu.VMEM((CHUNK, DEGW), jnp.float32),
        pltpu.VMEM((DEGCH, DEGW), jnp.float32),
        pltpu.VMEM_SHARED((N_U, DEGW), jnp.float32),
    ],
)(_deg_body)


def _nw_body(row2d, col2d, w2d, dinv_hbm, nw_out, dinvbuf, rbuf, cbuf, wbuf, obuf):
    c = lax.axis_index("c")
    s = lax.axis_index("s")
    wid = c * NS + s

    pltpu.sync_copy(dinv_hbm, dinvbuf)

    base = G // 32
    rem = G % 32
    start = wid * base + jnp.minimum(wid, rem)
    cnt = base + (wid < rem).astype(jnp.int32)

    def body(g, carry):
        pltpu.sync_copy(row2d.at[g], rbuf)
        pltpu.sync_copy(col2d.at[g], cbuf)
        pltpu.sync_copy(w2d.at[g], wbuf)
        for i in range(CHUNK // 16):
            sl = pl.ds(i * 16, 16)
            dr = plsc.load_gather(dinvbuf, [rbuf[sl]])
            dc = plsc.load_gather(dinvbuf, [cbuf[sl]])
            obuf[sl] = dr * wbuf[sl] * dc
        pltpu.sync_copy(obuf, nw_out.at[g])
        return carry

    lax.fori_loop(start, start + cnt, body, 0)


_nw_kernel = functools.partial(
    pl.kernel,
    out_type=jax.ShapeDtypeStruct((G, CHUNK), jnp.float32),
    mesh=_MESH,
    compiler_params=_CPARAMS,
    scratch_types=[
        pltpu.VMEM((N,), jnp.float32),
        pltpu.VMEM((CHUNK,), jnp.int32),
        pltpu.VMEM((CHUNK,), jnp.int32),
        pltpu.VMEM((CHUNK,), jnp.float32),
        pltpu.VMEM((CHUNK,), jnp.float32),
    ],
)(_nw_body)


def _layer_body(x_hbm, acc_hbm, col2d, rowloc2d, nw2d, x_out, acc_out,
                cbuf, rbuf, wbuf, rows, tbuf, abuf, zbuf, acc_sp, sem):
    c = lax.axis_index("c")
    s = lax.axis_index("s")

    _zero_zbuf(zbuf)
    for j in range(WOUT_IT):
        cid = j * NS + s
        @pl.when(cid < NROWCH)
        def _():
            pltpu.sync_copy(zbuf, acc_sp.at[pl.ds(cid * ROWCH, ROWCH)])
    plsc.subcore_barrier()

    start, cnt = _tile_range(c, s, GC, NS)

    def body(g, carry):
        pltpu.sync_copy(col2d.at[g], cbuf)
        pltpu.sync_copy(rowloc2d.at[g], rbuf)
        pltpu.sync_copy(nw2d.at[g], wbuf)
        pltpu.async_copy(x_hbm.at[cbuf], rows, sem).wait()

        def scale(i, carry2):
            w = plsc.load_gather(wbuf, [jnp.full((16,), i, jnp.int32)])
            for d in range(D // 16):
                sl = pl.ds(d * 16, 16)
                rows[i, sl] = rows[i, sl] * w
            return carry2

        lax.fori_loop(0, CHUNK, scale, 0)
        pltpu.sync_copy(rows, acc_sp.at[rbuf], add=True)
        return carry

    lax.fori_loop(start, start + cnt, body, 0)
    plsc.subcore_barrier()

    def wout(j, carry):
        cid = j * NS + s
        @pl.when(cid < NROWCH)
        def _():
            g0 = c * N_U + cid * ROWCH
            pltpu.sync_copy(acc_sp.at[pl.ds(cid * ROWCH, ROWCH)], tbuf)
            pltpu.sync_copy(acc_hbm.at[pl.ds(g0, ROWCH)], abuf)
            for jj in range(ROWCH):
                for d in range(D // 16):
                    sl = pl.ds(d * 16, 16)
                    abuf[jj, sl] = abuf[jj, sl] + tbuf[jj, sl]
            pltpu.sync_copy(tbuf, x_out.at[pl.ds(g0, ROWCH)])
            pltpu.sync_copy(abuf, acc_out.at[pl.ds(g0, ROWCH)])
        return carry

    lax.fori_loop(0, WOUT_IT, wout, 0)


_layer_kernel = functools.partial(
    pl.kernel,
    out_type=(jax.ShapeDtypeStruct((N, D), jnp.float32),
              jax.ShapeDtypeStruct((N, D), jnp.float32)),
    mesh=_MESH,
    compiler_params=_CPARAMS,
    scratch_types=[
        pltpu.VMEM((CHUNK,), jnp.int32),
        pltpu.VMEM((CHUNK,), jnp.int32),
        pltpu.VMEM((CHUNK,), jnp.float32),
        pltpu.VMEM((CHUNK, D), jnp.float32),
        pltpu.VMEM((ROWCH, D), jnp.float32),
        pltpu.VMEM((ROWCH, D), jnp.float32),
        pltpu.VMEM((ROWCH, D), jnp.float32),
        pltpu.VMEM_SHARED((N_U, D), jnp.float32),
        pltpu.SemaphoreType.DMA,
    ],
)(_layer_body)


def kernel(user_emb, item_emb, edge_index, edge_weight):
    row = edge_index[0]
    col = edge_index[1]
    row_local = jnp.where(row >= N_U, row - N_U, row)

    row2d = row.reshape(G, CHUNK)
    col2d = col.reshape(G, CHUNK)
    rowloc2d = row_local.reshape(G, CHUNK)
    w2d = edge_weight.reshape(G, CHUNK)

    deg16 = _deg_kernel(rowloc2d, w2d)
    dinv = lax.rsqrt(jnp.clip(deg16[:, 0], 1.0, None))
    nw2d = _nw_kernel(row2d, col2d, w2d, dinv)

    x0 = jnp.concatenate([user_emb, item_emb], axis=0)
    x1, a1 = _layer_kernel(x0, x0, col2d, rowloc2d, nw2d)
    x2, a2 = _layer_kernel(x1, a1, col2d, rowloc2d, nw2d)
    x3, a3 = _layer_kernel(x2, a2, col2d, rowloc2d, nw2d)

    final = a3 * 0.25
    return final[:N_U], final[N_U:]


# speedup vs baseline: 10.3462x; 1.2824x over previous
import functools

import jax
import jax.numpy as jnp
from jax import lax
from jax.experimental import pallas as pl
from jax.experimental.pallas import tpu as pltpu
from jax.experimental.pallas import tpu_sc as plsc

N_U = 25000
N_I = 25000
N = N_U + N_I
D = 64
E = 3200000 // 2
H = E // 2
CHUNK = 128
NS = 16
TCH = 393
P = NS * TCH
GP = 2 * P
HP = P * CHUNK
DEPTH = 3
ROWCH = 8
NROWCH = N_U // ROWCH
WOUT_IT = (NROWCH + NS - 1) // NS

_MESH = plsc.VectorSubcoreMesh(core_axis_name="c", subcore_axis_name="s")
_CPARAMS = pltpu.CompilerParams(needs_layout_passes=False,
                                use_tc_tiling_on_sc=False)


def _zero_zbuf(zbuf):
    z = jnp.zeros((16,), jnp.float32)
    for j in range(ROWCH):
        for d in range(D // 16):
            zbuf[j, pl.ds(d * 16, 16)] = z


DEGW = 16
DEGCH = 125


def _deg_body(rowloc2d, w2d, deg_out, idxbuf, wbuf, ubuf, tbuf, deg_sp):
    c = lax.axis_index("c")
    s = lax.axis_index("s")

    z = jnp.zeros((16,), jnp.float32)
    for j in range(DEGCH):
        tbuf[j, pl.ds(0, 16)] = z
    for j in range(13):
        cid = j * NS + s
        @pl.when(cid < 200)
        def _():
            pltpu.sync_copy(tbuf, deg_sp.at[pl.ds(cid * DEGCH, DEGCH)])
    for j in range(CHUNK):
        ubuf[j, pl.ds(0, 16)] = z
    plsc.subcore_barrier()

    start = c * P + s * TCH
    iota = lax.iota(jnp.int32, 16)
    czero = jnp.zeros((16,), jnp.int32)

    def body(j, carry):
        g = start + j
        pltpu.sync_copy(rowloc2d.at[g], idxbuf)
        pltpu.sync_copy(w2d.at[g], wbuf)
        for k in range(CHUNK // 16):
            plsc.store_scatter(ubuf, [iota + k * 16, czero], wbuf[pl.ds(k * 16, 16)])
        pltpu.sync_copy(ubuf, deg_sp.at[idxbuf], add=True)
        return carry

    lax.fori_loop(0, TCH, body, 0)
    plsc.subcore_barrier()

    for j in range(13):
        cid = j * NS + s
        @pl.when(cid < 200)
        def _():
            pltpu.sync_copy(deg_sp.at[pl.ds(cid * DEGCH, DEGCH)], tbuf)
            pltpu.sync_copy(tbuf, deg_out.at[pl.ds(c * N_U + cid * DEGCH, DEGCH)])


_deg_kernel = functools.partial(
    pl.kernel,
    out_type=jax.ShapeDtypeStruct((N, DEGW), jnp.float32),
    mesh=_MESH,
    compiler_params=_CPARAMS,
    scratch_types=[
        pltpu.VMEM((CHUNK,), jnp.int32),
        pltpu.VMEM((CHUNK,), jnp.float32),
        pltpu.VMEM((CHUNK, DEGW), jnp.float32),
        pltpu.VMEM((DEGCH, DEGW), jnp.float32),
        pltpu.VMEM_SHARED((N_U, DEGW), jnp.float32),
    ],
)(_deg_body)


def _nw_body(row2d, col2d, w2d, dinv_hbm, nw_out, dinvbuf, rbuf, cbuf, wbuf, obuf):
    c = lax.axis_index("c")
    s = lax.axis_index("s")
    wid = c * NS + s

    pltpu.sync_copy(dinv_hbm, dinvbuf)
    start = wid * TCH

    def body(j, carry):
        g = start + j
        pltpu.sync_copy(row2d.at[g], rbuf)
        pltpu.sync_copy(col2d.at[g], cbuf)
        pltpu.sync_copy(w2d.at[g], wbuf)
        for i in range(CHUNK // 16):
            sl = pl.ds(i * 16, 16)
            dr = plsc.load_gather(dinvbuf, [rbuf[sl]])
            dc = plsc.load_gather(dinvbuf, [cbuf[sl]])
            obuf[sl] = dr * wbuf[sl] * dc
        pltpu.sync_copy(obuf, nw_out.at[g])
        return carry

    lax.fori_loop(0, TCH, body, 0)


_nw_kernel = functools.partial(
    pl.kernel,
    out_type=jax.ShapeDtypeStruct((GP, CHUNK), jnp.float32),
    mesh=_MESH,
    compiler_params=_CPARAMS,
    scratch_types=[
        pltpu.VMEM((N,), jnp.float32),
        pltpu.VMEM((CHUNK,), jnp.int32),
        pltpu.VMEM((CHUNK,), jnp.int32),
        pltpu.VMEM((CHUNK,), jnp.float32),
        pltpu.VMEM((CHUNK,), jnp.float32),
    ],
)(_nw_body)


def _layer_body(x_hbm, acc_hbm, comb2, nw2d, x_out, acc_out,
                idx4, wv4, rows4, tbuf, abuf, zbuf, acc_sp,
                gs0, gs1, gs2, ss0, ss1, ss2):
    c = lax.axis_index("c")
    s = lax.axis_index("s")
    gsem = (gs0, gs1, gs2)
    ssem = (ss0, ss1, ss2)

    _zero_zbuf(zbuf)
    for j in range(WOUT_IT):
        cid = j * NS + s
        @pl.when(cid < NROWCH)
        def _():
            pltpu.sync_copy(zbuf, acc_sp.at[pl.ds(cid * ROWCH, ROWCH)])
    plsc.subcore_barrier()

    base = c * P + s * TCH

    def drain(k, sem):
        pltpu.make_async_copy(
            x_hbm.at[pl.ds(0, CHUNK)], rows4.at[k], sem).wait()

    def prep(n, k):
        @pl.when(n >= DEPTH)
        def _():
            drain(k, ssem[k])
        pltpu.sync_copy(comb2.at[base + n], idx4.at[k])
        pltpu.sync_copy(nw2d.at[base + n], wv4.at[k])
        pltpu.async_copy(x_hbm.at[idx4.at[k, 0]], rows4.at[k], gsem[k])

    def finish(k):
        drain(k, gsem[k])

        ksp = jnp.full((16,), k, jnp.int32)

        def scale(i, carry):
            w = plsc.load_gather(wv4, [ksp, jnp.full((16,), i, jnp.int32)])
            for d in range(D // 16):
                sl = pl.ds(d * 16, 16)
                rows4[k, i, sl] = rows4[k, i, sl] * w
            return carry

        lax.fori_loop(0, CHUNK, scale, 0, unroll=2)
        pltpu.async_copy(rows4.at[k], acc_sp.at[idx4.at[k, 1]], ssem[k],
                         add=True)

    prep(0, 0)
    prep(1, 1)

    def macro(m, carry):
        for k in range(DEPTH):
            g = m * DEPTH + k
            n = g + 2
            @pl.when(n < TCH)
            def _():
                prep(n, (k + 2) % DEPTH)
            finish(k)
        return carry

    lax.fori_loop(0, TCH // DEPTH, macro, 0)
    for k in range(DEPTH):
        drain(k, ssem[k])
    plsc.subcore_barrier()

    def wout(j, carry):
        cid = j * NS + s
        @pl.when(cid < NROWCH)
        def _():
            g0 = c * N_U + cid * ROWCH
            pltpu.sync_copy(acc_sp.at[pl.ds(cid * ROWCH, ROWCH)], tbuf)
            pltpu.sync_copy(acc_hbm.at[pl.ds(g0, ROWCH)], abuf)
            for jj in range(ROWCH):
                for d in range(D // 16):
                    sl = pl.ds(d * 16, 16)
                    abuf[jj, sl] = abuf[jj, sl] + tbuf[jj, sl]
            pltpu.sync_copy(tbuf, x_out.at[pl.ds(g0, ROWCH)])
            pltpu.sync_copy(abuf, acc_out.at[pl.ds(g0, ROWCH)])
        return carry

    lax.fori_loop(0, WOUT_IT, wout, 0)


_layer_kernel = functools.partial(
    pl.kernel,
    out_type=(jax.ShapeDtypeStruct((N, D), jnp.float32),
              jax.ShapeDtypeStruct((N, D), jnp.float32)),
    mesh=_MESH,
    compiler_params=_CPARAMS,
    scratch_types=[
        pltpu.VMEM((DEPTH, 2, CHUNK), jnp.int32),
        pltpu.VMEM((DEPTH, CHUNK), jnp.float32),
        pltpu.VMEM((DEPTH, CHUNK, D), jnp.float32),
        pltpu.VMEM((ROWCH, D), jnp.float32),
        pltpu.VMEM((ROWCH, D), jnp.float32),
        pltpu.VMEM((ROWCH, D), jnp.float32),
        pltpu.VMEM_SHARED((N_U, D), jnp.float32),
        pltpu.SemaphoreType.DMA,
        pltpu.SemaphoreType.DMA,
        pltpu.SemaphoreType.DMA,
        pltpu.SemaphoreType.DMA,
        pltpu.SemaphoreType.DMA,
        pltpu.SemaphoreType.DMA,
    ],
)(_layer_body)


def _pad_half(a, fill):
    pad = jnp.full((HP - H,), fill, a.dtype)
    return jnp.concatenate([a[:H], pad, a[H:], pad])


def kernel(user_emb, item_emb, edge_index, edge_weight):
    row = edge_index[0]
    col = edge_index[1]
    row_local = jnp.where(row >= N_U, row - N_U, row)

    rowp2d = _pad_half(row, 0).reshape(GP, CHUNK)
    colp2d = _pad_half(col, 0).reshape(GP, CHUNK)
    rowlocp2d = _pad_half(row_local, 0).reshape(GP, CHUNK)
    wp2d = _pad_half(edge_weight, 0.0).reshape(GP, CHUNK)
    comb2 = jnp.stack([colp2d, rowlocp2d], axis=1)

    deg16 = _deg_kernel(rowlocp2d, wp2d)
    dinv = lax.rsqrt(jnp.clip(deg16[:, 0], 1.0, None))
    nw2d = _nw_kernel(rowp2d, colp2d, wp2d, dinv)

    x0 = jnp.concatenate([user_emb, item_emb], axis=0)
    x1, a1 = _layer_kernel(x0, x0, comb2, nw2d)
    x2, a2 = _layer_kernel(x1, a1, comb2, nw2d)
    x3, a3 = _layer_kernel(x2, a2, comb2, nw2d)

    final = a3 * 0.25
    return final[:N_U], final[N_U:]


# speedup vs baseline: 11.8825x; 1.1485x over previous
import functools

import jax
import jax.numpy as jnp
from jax import lax
from jax.experimental import pallas as pl
from jax.experimental.pallas import tpu as pltpu
from jax.experimental.pallas import tpu_sc as plsc

N_U = 25000
N_I = 25000
N = N_U + N_I
D = 64
E = 3200000 // 2
H = E // 2
CHUNK = 128
NS = 16
TCH = 396
P = NS * TCH
GP = 2 * P
HP = P * CHUNK
DEPTH = 3
ROWCH = 8
NROWCH = N_U // ROWCH
WOUT_IT = (NROWCH + NS - 1) // NS

_MESH = plsc.VectorSubcoreMesh(core_axis_name="c", subcore_axis_name="s")
_CPARAMS = pltpu.CompilerParams(needs_layout_passes=False,
                                use_tc_tiling_on_sc=False)


def _zero_zbuf(zbuf):
    z = jnp.zeros((16,), jnp.float32)
    for j in range(ROWCH):
        for d in range(D // 16):
            zbuf[j, pl.ds(d * 16, 16)] = z


DEGW = 16
DEGCH = 125


def _deg_body(rowloc2d, w2d, deg_out, idxbuf, wbuf, ubuf, tbuf, deg_sp):
    c = lax.axis_index("c")
    s = lax.axis_index("s")

    z = jnp.zeros((16,), jnp.float32)
    for j in range(DEGCH):
        tbuf[j, pl.ds(0, 16)] = z
    for j in range(13):
        cid = j * NS + s
        @pl.when(cid < 200)
        def _():
            pltpu.sync_copy(tbuf, deg_sp.at[pl.ds(cid * DEGCH, DEGCH)])
    for j in range(CHUNK):
        ubuf[j, pl.ds(0, 16)] = z
    plsc.subcore_barrier()

    start = c * P + s * TCH
    iota = lax.iota(jnp.int32, 16)
    czero = jnp.zeros((16,), jnp.int32)

    def body(j, carry):
        g = start + j
        pltpu.sync_copy(rowloc2d.at[g], idxbuf)
        pltpu.sync_copy(w2d.at[g], wbuf)
        for k in range(CHUNK // 16):
            plsc.store_scatter(ubuf, [iota + k * 16, czero], wbuf[pl.ds(k * 16, 16)])
        pltpu.sync_copy(ubuf, deg_sp.at[idxbuf], add=True)
        return carry

    lax.fori_loop(0, TCH, body, 0)
    plsc.subcore_barrier()

    for j in range(13):
        cid = j * NS + s
        @pl.when(cid < 200)
        def _():
            pltpu.sync_copy(deg_sp.at[pl.ds(cid * DEGCH, DEGCH)], tbuf)
            pltpu.sync_copy(tbuf, deg_out.at[pl.ds(c * N_U + cid * DEGCH, DEGCH)])


_deg_kernel = functools.partial(
    pl.kernel,
    out_type=jax.ShapeDtypeStruct((N, DEGW), jnp.float32),
    mesh=_MESH,
    compiler_params=_CPARAMS,
    scratch_types=[
        pltpu.VMEM((CHUNK,), jnp.int32),
        pltpu.VMEM((CHUNK,), jnp.float32),
        pltpu.VMEM((CHUNK, DEGW), jnp.float32),
        pltpu.VMEM((DEGCH, DEGW), jnp.float32),
        pltpu.VMEM_SHARED((N_U, DEGW), jnp.float32),
    ],
)(_deg_body)


def _nw_body(row2d, col2d, w2d, dinv_hbm, nw_out, dinvbuf, rbuf, cbuf, wbuf, obuf):
    c = lax.axis_index("c")
    s = lax.axis_index("s")
    wid = c * NS + s

    pltpu.sync_copy(dinv_hbm, dinvbuf)
    start = wid * TCH

    def body(j, carry):
        g = start + j
        pltpu.sync_copy(row2d.at[g], rbuf)
        pltpu.sync_copy(col2d.at[g], cbuf)
        pltpu.sync_copy(w2d.at[g], wbuf)
        for i in range(CHUNK // 16):
            sl = pl.ds(i * 16, 16)
            dr = plsc.load_gather(dinvbuf, [rbuf[sl]])
            dc = plsc.load_gather(dinvbuf, [cbuf[sl]])
            obuf[sl] = plsc.bitcast(dr * wbuf[sl] * dc, jnp.int32)
        pltpu.sync_copy(obuf, nw_out.at[g])
        return carry

    lax.fori_loop(0, TCH, body, 0)


_nw_kernel = functools.partial(
    pl.kernel,
    out_type=jax.ShapeDtypeStruct((GP, CHUNK), jnp.int32),
    mesh=_MESH,
    compiler_params=_CPARAMS,
    scratch_types=[
        pltpu.VMEM((N,), jnp.float32),
        pltpu.VMEM((CHUNK,), jnp.int32),
        pltpu.VMEM((CHUNK,), jnp.int32),
        pltpu.VMEM((CHUNK,), jnp.float32),
        pltpu.VMEM((CHUNK,), jnp.int32),
    ],
)(_nw_body)


ID = 6
RD = 3


def _layer_body(x_hbm, acc_hbm, comb3, x_out, acc_out,
                idx6, rows3, tbuf, abuf, zbuf, acc_sp,
                is0, is1, is2, is3, is4, is5, gs0, gs1, gs2, ss0, ss1, ss2):
    c = lax.axis_index("c")
    s = lax.axis_index("s")
    isem = (is0, is1, is2, is3, is4, is5)
    gsem = (gs0, gs1, gs2)
    ssem = (ss0, ss1, ss2)

    _zero_zbuf(zbuf)
    for j in range(WOUT_IT):
        cid = j * NS + s
        @pl.when(cid < NROWCH)
        def _():
            pltpu.sync_copy(zbuf, acc_sp.at[pl.ds(cid * ROWCH, ROWCH)])
    plsc.subcore_barrier()

    base = c * P + s * TCH

    def idx_issue(n, k):
        pltpu.async_copy(comb3.at[base + n], idx6.at[k], isem[k])

    def idx_drain(k):
        pltpu.make_async_copy(comb3.at[0], idx6.at[k], isem[k]).wait()

    def row_drain(r, sem):
        pltpu.make_async_copy(x_hbm.at[pl.ds(0, CHUNK)], rows3.at[r], sem).wait()

    def gather_issue(k, r):
        pltpu.async_copy(x_hbm.at[idx6.at[k, 0]], rows3.at[r], gsem[r])

    def finish(g, k, r):
        row_drain(r, gsem[r])
        ksp = jnp.full((16,), k, jnp.int32)
        two = jnp.full((16,), 2, jnp.int32)

        def scale(i, carry):
            w = plsc.bitcast(
                plsc.load_gather(idx6, [ksp, two, jnp.full((16,), i, jnp.int32)]),
                jnp.float32)
            for d in range(D // 16):
                sl = pl.ds(d * 16, 16)
                rows3[r, i, sl] = rows3[r, i, sl] * w
            return carry

        lax.fori_loop(0, CHUNK, scale, 0, unroll=4)
        pltpu.async_copy(rows3.at[r], acc_sp.at[idx6.at[k, 1]], ssem[r],
                         add=True)

    pltpu.sync_copy(comb3.at[base], idx6.at[0])
    gather_issue(0, 0)
    idx_issue(1, 1)
    idx_issue(2, 2)

    def macro(m, carry):
        for k in range(ID):
            g = m * ID + k
            r = k % RD
            r1 = (k + 1) % RD
            @pl.when(g + 3 < TCH)
            def _():
                idx_issue(g + 3, (k + 3) % ID)
            @pl.when(g >= 2)
            def _():
                row_drain(r1, ssem[r1])
            @pl.when(g + 1 < TCH)
            def _():
                idx_drain((k + 1) % ID)
                gather_issue((k + 1) % ID, r1)
            finish(g, k, r)
        return carry

    lax.fori_loop(0, TCH // ID, macro, 0)
    row_drain(1, ssem[1])
    row_drain(2, ssem[2])
    plsc.subcore_barrier()

    def wout(j, carry):
        cid = j * NS + s
        @pl.when(cid < NROWCH)
        def _():
            g0 = c * N_U + cid * ROWCH
            pltpu.sync_copy(acc_sp.at[pl.ds(cid * ROWCH, ROWCH)], tbuf)
            pltpu.sync_copy(acc_hbm.at[pl.ds(g0, ROWCH)], abuf)
            for jj in range(ROWCH):
                for d in range(D // 16):
                    sl = pl.ds(d * 16, 16)
                    abuf[jj, sl] = abuf[jj, sl] + tbuf[jj, sl]
            pltpu.sync_copy(tbuf, x_out.at[pl.ds(g0, ROWCH)])
            pltpu.sync_copy(abuf, acc_out.at[pl.ds(g0, ROWCH)])
        return carry

    lax.fori_loop(0, WOUT_IT, wout, 0)


_layer_kernel = functools.partial(
    pl.kernel,
    out_type=(jax.ShapeDtypeStruct((N, D), jnp.float32),
              jax.ShapeDtypeStruct((N, D), jnp.float32)),
    mesh=_MESH,
    compiler_params=_CPARAMS,
    scratch_types=[
        pltpu.VMEM((ID, 3, CHUNK), jnp.int32),
        pltpu.VMEM((RD, CHUNK, D), jnp.float32),
        pltpu.VMEM((ROWCH, D), jnp.float32),
        pltpu.VMEM((ROWCH, D), jnp.float32),
        pltpu.VMEM((ROWCH, D), jnp.float32),
        pltpu.VMEM_SHARED((N_U, D), jnp.float32),
    ] + [pltpu.SemaphoreType.DMA] * 12,
)(_layer_body)


def _pad_half(a, fill):
    pad = jnp.full((HP - H,), fill, a.dtype)
    return jnp.concatenate([a[:H], pad, a[H:], pad])


def kernel(user_emb, item_emb, edge_index, edge_weight):
    row = edge_index[0]
    col = edge_index[1]
    row_local = jnp.where(row >= N_U, row - N_U, row)

    rowp2d = _pad_half(row, 0).reshape(GP, CHUNK)
    colp2d = _pad_half(col, 0).reshape(GP, CHUNK)
    rowlocp2d = _pad_half(row_local, 0).reshape(GP, CHUNK)
    wp2d = _pad_half(edge_weight, 0.0).reshape(GP, CHUNK)

    deg16 = _deg_kernel(rowlocp2d, wp2d)
    dinv = lax.rsqrt(jnp.clip(deg16[:, 0], 1.0, None))
    nw2d = _nw_kernel(rowp2d, colp2d, wp2d, dinv)
    comb3 = jnp.stack([colp2d, rowlocp2d, nw2d], axis=1)

    x0 = jnp.concatenate([user_emb, item_emb], axis=0)
    x1, a1 = _layer_kernel(x0, x0, comb3)
    x2, a2 = _layer_kernel(x1, a1, comb3)
    x3, a3 = _layer_kernel(x2, a2, comb3)

    final = a3 * 0.25
    return final[:N_U], final[N_U:]


# speedup vs baseline: 14.5697x; 1.2261x over previous
import functools

import jax
import jax.numpy as jnp
from jax import lax
from jax.experimental import pallas as pl
from jax.experimental.pallas import tpu as pltpu
from jax.experimental.pallas import tpu_sc as plsc

N_U = 25000
N_I = 25000
N = N_U + N_I
D = 64
E = 3200000 // 2
H = E // 2
CHUNK = 128
NS = 16
TCH = 396
P = NS * TCH
GP = 2 * P
HP = P * CHUNK
DEPTH = 3
ROWCH = 8
NROWCH = N_U // ROWCH
WOUT_IT = (NROWCH + NS - 1) // NS

_MESH = plsc.VectorSubcoreMesh(core_axis_name="c", subcore_axis_name="s")
_CPARAMS = pltpu.CompilerParams(needs_layout_passes=False,
                                use_tc_tiling_on_sc=False)


def _zero_zbuf(zbuf):
    z = jnp.zeros((16,), jnp.float32)
    for j in range(ROWCH):
        for d in range(D // 16):
            zbuf[j, pl.ds(d * 16, 16)] = z


DEGW = 16
DEGCH = 125


def _deg_body(rcw3, deg_out, ebuf, idxbuf, ubuf, tbuf, deg_sp, esem):
    c = lax.axis_index("c")
    s = lax.axis_index("s")

    z = jnp.zeros((16,), jnp.float32)
    for j in range(DEGCH):
        tbuf[j, pl.ds(0, 16)] = z
    for j in range(13):
        cid = j * NS + s
        @pl.when(cid < 200)
        def _():
            pltpu.sync_copy(tbuf, deg_sp.at[pl.ds(cid * DEGCH, DEGCH)])
    for j in range(CHUNK):
        ubuf[j, pl.ds(0, 16)] = z
    plsc.subcore_barrier()

    start = c * P + s * TCH
    iota = lax.iota(jnp.int32, 16)
    czero = jnp.zeros((16,), jnp.int32)
    nu = jnp.full((16,), N_U, jnp.int32)

    pltpu.sync_copy(rcw3.at[start], ebuf.at[0])
    pltpu.async_copy(rcw3.at[start + 1], ebuf.at[1], esem)

    def body(j, carry):
        b = j % 2
        @pl.when(j >= 1)
        def _():
            pltpu.make_async_copy(rcw3.at[0], ebuf.at[b], esem).wait()
        for k in range(CHUNK // 16):
            sl = pl.ds(k * 16, 16)
            r = ebuf[b, 0, sl]
            rl = jnp.where(r >= nu, r - nu, r)
            idxbuf[sl] = rl
            plsc.store_scatter(
                ubuf, [iota + k * 16, czero],
                plsc.bitcast(ebuf[b, 2, sl], jnp.float32))
        pltpu.sync_copy(ubuf, deg_sp.at[idxbuf], add=True)
        @pl.when(j + 2 < TCH)
        def _():
            pltpu.async_copy(rcw3.at[start + j + 2], ebuf.at[b], esem)
        return carry

    lax.fori_loop(0, TCH, body, 0)
    plsc.subcore_barrier()

    for j in range(13):
        cid = j * NS + s
        @pl.when(cid < 200)
        def _():
            pltpu.sync_copy(deg_sp.at[pl.ds(cid * DEGCH, DEGCH)], tbuf)
            pltpu.sync_copy(tbuf, deg_out.at[pl.ds(c * N_U + cid * DEGCH, DEGCH)])


_deg_kernel = functools.partial(
    pl.kernel,
    out_type=jax.ShapeDtypeStruct((N, DEGW), jnp.float32),
    mesh=_MESH,
    compiler_params=_CPARAMS,
    scratch_types=[
        pltpu.VMEM((2, 3, CHUNK), jnp.int32),
        pltpu.VMEM((CHUNK,), jnp.int32),
        pltpu.VMEM((CHUNK, DEGW), jnp.float32),
        pltpu.VMEM((DEGCH, DEGW), jnp.float32),
        pltpu.VMEM_SHARED((N_U, DEGW), jnp.float32),
        pltpu.SemaphoreType.DMA,
    ],
)(_deg_body)


def _nw_body(rcw3, dinv_hbm, comb_out, dinvbuf, ebuf, obuf, esem, osem):
    c = lax.axis_index("c")
    s = lax.axis_index("s")
    wid = c * NS + s

    pltpu.sync_copy(dinv_hbm, dinvbuf)
    start = wid * TCH
    nu = jnp.full((16,), N_U, jnp.int32)

    pltpu.sync_copy(rcw3.at[start], ebuf.at[0])
    pltpu.async_copy(rcw3.at[start + 1], ebuf.at[1], esem)

    def body(j, carry):
        b = j % 2
        @pl.when(j >= 2)
        def _():
            pltpu.make_async_copy(rcw3.at[0], obuf.at[b], osem).wait()
        @pl.when(j >= 1)
        def _():
            pltpu.make_async_copy(rcw3.at[0], ebuf.at[b], esem).wait()
        for i in range(CHUNK // 16):
            sl = pl.ds(i * 16, 16)
            r = ebuf[b, 0, sl]
            col = ebuf[b, 1, sl]
            w = plsc.bitcast(ebuf[b, 2, sl], jnp.float32)
            dr = plsc.load_gather(dinvbuf, [r])
            dc = plsc.load_gather(dinvbuf, [col])
            obuf[b, 0, sl] = col
            obuf[b, 1, sl] = jnp.where(r >= nu, r - nu, r)
            obuf[b, 2, sl] = plsc.bitcast(dr * w * dc, jnp.int32)
        pltpu.async_copy(obuf.at[b], comb_out.at[start + j], osem)
        @pl.when(j + 2 < TCH)
        def _():
            pltpu.async_copy(rcw3.at[start + j + 2], ebuf.at[b], esem)
        return carry

    lax.fori_loop(0, TCH, body, 0)
    pltpu.make_async_copy(rcw3.at[0], obuf.at[0], osem).wait()
    pltpu.make_async_copy(rcw3.at[0], obuf.at[1], osem).wait()


_nw_kernel = functools.partial(
    pl.kernel,
    out_type=jax.ShapeDtypeStruct((GP, 3, CHUNK), jnp.int32),
    mesh=_MESH,
    compiler_params=_CPARAMS,
    scratch_types=[
        pltpu.VMEM((N,), jnp.float32),
        pltpu.VMEM((2, 3, CHUNK), jnp.int32),
        pltpu.VMEM((2, 3, CHUNK), jnp.int32),
        pltpu.SemaphoreType.DMA,
        pltpu.SemaphoreType.DMA,
    ],
)(_nw_body)


ID = 6
RD = 3


def _layer_body(x_hbm, acc_hbm, comb3, x_out, acc_out,
                idx6, rows3, tbuf, abuf, zbuf, acc_sp,
                is0, is1, is2, is3, is4, is5, gs0, gs1, gs2, ss0, ss1, ss2):
    c = lax.axis_index("c")
    s = lax.axis_index("s")
    isem = (is0, is1, is2, is3, is4, is5)
    gsem = (gs0, gs1, gs2)
    ssem = (ss0, ss1, ss2)

    _zero_zbuf(zbuf)
    for j in range(WOUT_IT):
        cid = j * NS + s
        @pl.when(cid < NROWCH)
        def _():
            pltpu.sync_copy(zbuf, acc_sp.at[pl.ds(cid * ROWCH, ROWCH)])
    plsc.subcore_barrier()

    base = c * P + s * TCH

    def idx_issue(n, k):
        pltpu.async_copy(comb3.at[base + n], idx6.at[k], isem[k])

    def idx_drain(k):
        pltpu.make_async_copy(comb3.at[0], idx6.at[k], isem[k]).wait()

    def row_drain(r, sem):
        pltpu.make_async_copy(x_hbm.at[pl.ds(0, CHUNK)], rows3.at[r], sem).wait()

    def gather_issue(k, r):
        pltpu.async_copy(x_hbm.at[idx6.at[k, 0]], rows3.at[r], gsem[r])

    def finish(g, k, r):
        row_drain(r, gsem[r])
        ksp = jnp.full((16,), k, jnp.int32)
        two = jnp.full((16,), 2, jnp.int32)

        def scale(i, carry):
            w = plsc.bitcast(
                plsc.load_gather(idx6, [ksp, two, jnp.full((16,), i, jnp.int32)]),
                jnp.float32)
            for d in range(D // 16):
                sl = pl.ds(d * 16, 16)
                rows3[r, i, sl] = rows3[r, i, sl] * w
            return carry

        lax.fori_loop(0, CHUNK, scale, 0, unroll=4)
        pltpu.async_copy(rows3.at[r], acc_sp.at[idx6.at[k, 1]], ssem[r],
                         add=True)

    pltpu.sync_copy(comb3.at[base], idx6.at[0])
    gather_issue(0, 0)
    idx_issue(1, 1)
    idx_issue(2, 2)

    def macro(m, carry):
        for k in range(ID):
            g = m * ID + k
            r = k % RD
            r1 = (k + 1) % RD
            @pl.when(g + 3 < TCH)
            def _():
                idx_issue(g + 3, (k + 3) % ID)
            @pl.when(g >= 2)
            def _():
                row_drain(r1, ssem[r1])
            @pl.when(g + 1 < TCH)
            def _():
                idx_drain((k + 1) % ID)
                gather_issue((k + 1) % ID, r1)
            finish(g, k, r)
        return carry

    lax.fori_loop(0, TCH // ID, macro, 0)
    row_drain(1, ssem[1])
    row_drain(2, ssem[2])
    plsc.subcore_barrier()

    def wout(j, carry):
        cid = j * NS + s
        @pl.when(cid < NROWCH)
        def _():
            g0 = c * N_U + cid * ROWCH
            pltpu.sync_copy(acc_sp.at[pl.ds(cid * ROWCH, ROWCH)], tbuf)
            pltpu.sync_copy(acc_hbm.at[pl.ds(g0, ROWCH)], abuf)
            for jj in range(ROWCH):
                for d in range(D // 16):
                    sl = pl.ds(d * 16, 16)
                    abuf[jj, sl] = abuf[jj, sl] + tbuf[jj, sl]
            pltpu.sync_copy(tbuf, x_out.at[pl.ds(g0, ROWCH)])
            pltpu.sync_copy(abuf, acc_out.at[pl.ds(g0, ROWCH)])
        return carry

    lax.fori_loop(0, WOUT_IT, wout, 0)


_layer_kernel = functools.partial(
    pl.kernel,
    out_type=(jax.ShapeDtypeStruct((N, D), jnp.float32),
              jax.ShapeDtypeStruct((N, D), jnp.float32)),
    mesh=_MESH,
    compiler_params=_CPARAMS,
    scratch_types=[
        pltpu.VMEM((ID, 3, CHUNK), jnp.int32),
        pltpu.VMEM((RD, CHUNK, D), jnp.float32),
        pltpu.VMEM((ROWCH, D), jnp.float32),
        pltpu.VMEM((ROWCH, D), jnp.float32),
        pltpu.VMEM((ROWCH, D), jnp.float32),
        pltpu.VMEM_SHARED((N_U, D), jnp.float32),
    ] + [pltpu.SemaphoreType.DMA] * 12,
)(_layer_body)


def _pad_half(a, fill):
    pad = jnp.full((HP - H,), fill, a.dtype)
    return jnp.concatenate([a[:H], pad, a[H:], pad])


def kernel(user_emb, item_emb, edge_index, edge_weight):
    row = edge_index[0]
    col = edge_index[1]

    rowp2d = _pad_half(row, 0).reshape(GP, CHUNK)
    colp2d = _pad_half(col, 0).reshape(GP, CHUNK)
    wp2d = _pad_half(edge_weight, 0.0).reshape(GP, CHUNK)
    rcw3 = jnp.stack([rowp2d, colp2d,
                      lax.bitcast_convert_type(wp2d, jnp.int32)], axis=1)

    deg16 = _deg_kernel(rcw3)
    dinv = lax.rsqrt(jnp.clip(deg16[:, 0], 1.0, None))
    comb3 = _nw_kernel(rcw3, dinv)

    x0 = jnp.concatenate([user_emb, item_emb], axis=0)
    x1, a1 = _layer_kernel(x0, x0, comb3)
    x2, a2 = _layer_kernel(x1, a1, comb3)
    x3, a3 = _layer_kernel(x2, a2, comb3)

    final = a3 * 0.25
    return final[:N_U], final[N_U:]
